# baseline (speedup 1.0000x reference)
import functools

import jax
import jax.numpy as jnp
from jax import lax
from jax.experimental import pallas as pl
from jax.experimental.pallas import tpu as pltpu
from jax.experimental.pallas import tpu_sc as plsc

B = 1024
M = 100000
D = 256
K = 32
W = 128
MSTEP = 4096
NSTEP = (M + MSTEP - 1) // MSTEP
MPAD = NSTEP * MSTEP
NBLK = MPAD // W
GPB = MSTEP // W
NEG = -3.4e38
LAMBDAH = 0.3


def _score_body(q_ref, k_ref, s_ref, bm_ref):
    j = pl.program_id(0)
    q = q_ref[...]
    keys = k_ref[...]
    dots = lax.dot_general(q, keys, (((1,), (1,)), ((), ())),
                           preferred_element_type=jnp.float32)
    ones8 = jnp.ones((8, D), jnp.float32)
    s2 = lax.dot_general(ones8, keys * keys, (((1,), (1,)), ((), ())),
                         preferred_element_type=jnp.float32)[0:1]
    col = j * MSTEP + lax.broadcasted_iota(jnp.int32, (1, MSTEP), 1)
    score = dots - 0.5 * s2
    score = jnp.where(col >= M, NEG, score)
    s3 = score.reshape(B, GPB, W)
    s_ref[...] = s3
    bm_ref[...] = jnp.max(s3, axis=2)[None]


def _scores_fn(query, mem_keys, interpret=False):
    return pl.pallas_call(
        _score_body,
        grid=(NSTEP,),
        in_specs=[
            pl.BlockSpec((B, D), lambda j: (0, 0)),
            pl.BlockSpec((MSTEP, D), lambda j: (j, 0)),
        ],
        out_specs=[
            pl.BlockSpec((B, GPB, W), lambda j: (0, j, 0)),
            pl.BlockSpec((1, B, GPB), lambda j: (j, 0, 0)),
        ],
        out_shape=[
            jax.ShapeDtypeStruct((B, NBLK, W), jnp.float32),
            jax.ShapeDtypeStruct((NSTEP, B, GPB), jnp.float32),
        ],
        interpret=interpret,
    )(query, mem_keys)


def _topblocks_body(bm_ref, gidx_ref):
    bm = bm_ref[...]
    iota = lax.broadcasted_iota(jnp.int32, (B, NBLK), 1)
    row = lax.broadcasted_iota(jnp.int32, (B, 1), 0)
    for t in range(K):
        m = jnp.max(bm, axis=1, keepdims=True)
        pos = jnp.min(jnp.where(bm == m, iota, jnp.int32(1 << 30)),
                      axis=1, keepdims=True)
        bm = jnp.where(iota == pos, NEG, bm)
        gidx_ref[:, t:t + 1] = row * NBLK + pos
    del bm


def _topblocks_fn(bmax, interpret=False):
    return pl.pallas_call(
        _topblocks_body,
        out_shape=jax.ShapeDtypeStruct((B, K), jnp.int32),
        interpret=interpret,
    )(bmax)


def _sc_gather(table, idx, rows, d):
    n = idx.shape[0]
    info = plsc.get_sparse_core_info()
    nw = info.num_cores * info.num_subcores
    per_w = n // nw
    chunk = 128 if per_w % 128 == 0 else per_w
    nchunk = per_w // chunk
    mesh = plsc.VectorSubcoreMesh(core_axis_name="c", subcore_axis_name="s")

    @functools.partial(
        pl.kernel, mesh=mesh,
        out_type=jax.ShapeDtypeStruct((n, d), jnp.float32),
        scratch_types=[
            pltpu.VMEM((chunk,), jnp.int32),
            pltpu.VMEM((chunk, d), jnp.float32),
            pltpu.SemaphoreType.DMA,
        ],
    )
    def k(table_hbm, idx_hbm, out_hbm, idx_v, rows_v, sem):
        wid = lax.axis_index("s") * info.num_cores + lax.axis_index("c")
        base = wid * per_w

        def body(i, carry):
            off = base + i * chunk
            pltpu.sync_copy(idx_hbm.at[pl.ds(off, chunk)], idx_v)
            pltpu.async_copy(table_hbm.at[idx_v], rows_v, sem).wait()
            pltpu.sync_copy(rows_v, out_hbm.at[pl.ds(off, chunk)])
            return carry

        lax.fori_loop(0, nchunk, body, 0)

    return k(table, idx)


def _extract_body(c_ref, gidx_ref, q_ref, ti_ref, w_ref):
    c = c_ref[...]
    gidx = gidx_ref[...]
    q = q_ref[...]
    rb = c.shape[0]
    iota = lax.broadcasted_iota(jnp.int32, (rb, K * W), 1)
    kiota = lax.broadcasted_iota(jnp.int32, (rb, K), 1)
    row = lax.broadcasted_iota(jnp.int32, (rb, 1), 0)
    blkid = gidx - (pl.program_id(0) * rb + row) * NBLK
    scores = []
    for t in range(K):
        m = jnp.max(c, axis=1, keepdims=True)
        pos = jnp.min(jnp.where(c == m, iota, jnp.int32(1 << 30)),
                      axis=1, keepdims=True)
        c = jnp.where(iota == pos, NEG, c)
        g = jnp.right_shift(pos, 7)
        off = jnp.bitwise_and(pos, jnp.int32(W - 1))
        blk = jnp.sum(jnp.where(kiota == g, blkid, 0), axis=1, keepdims=True)
        ti_ref[:, t:t + 1] = blk * W + off
        scores.append(m)
    ts = jnp.concatenate(scores, axis=1)
    q2 = jnp.sum(q * q, axis=1, keepdims=True)
    sim = jnp.exp(ts - 0.5 * q2)
    w_ref[...] = sim / (jnp.sum(sim, axis=1, keepdims=True) + 1e-8)


def _extract_fn(cand, gidx, query, interpret=False):
    rb = 256
    nb = B // rb
    return pl.pallas_call(
        _extract_body,
        grid=(nb,),
        in_specs=[
            pl.BlockSpec((rb, K * W), lambda i: (i, 0)),
            pl.BlockSpec((rb, K), lambda i: (i, 0)),
            pl.BlockSpec((rb, D), lambda i: (i, 0)),
        ],
        out_specs=[
            pl.BlockSpec((rb, K), lambda i: (i, 0)),
            pl.BlockSpec((rb, K), lambda i: (i, 0)),
        ],
        out_shape=[
            jax.ShapeDtypeStruct((B, K), jnp.int32),
            jax.ShapeDtypeStruct((B, K), jnp.float32),
        ],
        interpret=interpret,
    )(cand, gidx, query)


def _combine_body(v_ref, w_ref, np_ref, o_ref):
    vals = v_ref[...]
    w = w_ref[...]
    pred = jnp.sum(vals * w[:, :, None], axis=1)
    o_ref[...] = LAMBDAH * pred + (1.0 - LAMBDAH) * np_ref[...]


def _combine_fn(vals, w, next_pred, interpret=False):
    rb = 128
    nb = B // rb
    return pl.pallas_call(
        _combine_body,
        grid=(nb,),
        in_specs=[
            pl.BlockSpec((rb, K, D), lambda i: (i, 0, 0)),
            pl.BlockSpec((rb, K), lambda i: (i, 0)),
            pl.BlockSpec((rb, D), lambda i: (i, 0)),
        ],
        out_specs=pl.BlockSpec((rb, D), lambda i: (i, 0)),
        out_shape=jax.ShapeDtypeStruct((B, D), jnp.float32),
        interpret=interpret,
    )(vals, w, next_pred)


def kernel(query, mem_keys, mem_values, next_pred, k):
    scores, bmax3 = _scores_fn(query, mem_keys)
    bmax = bmax3.transpose(1, 0, 2).reshape(B, NBLK)
    gidx = _topblocks_fn(bmax)
    cand = _sc_gather(scores.reshape(B * NBLK, W), gidx.reshape(B * K), B * NBLK, W)
    top_idx, w = _extract_fn(cand.reshape(B, K * W), gidx, query)
    vals = _sc_gather(mem_values, top_idx.reshape(B * K), M, D)
    return _combine_fn(vals.reshape(B, K, D), w, next_pred)
    bmax = bmax3.transpose(1, 0, 2).reshape(B, NBLK)
    gidx = _topblocks_fn(bmax)
    cand = _sc_gather(scores.reshape(B * NBLK, W), gidx.reshape(B * K), B * NBLK, W)
    top_idx, w = _extract_fn(cand.reshape(B, K * W), gidx, query)
    vals = _sc_gather(mem_values, top_idx.reshape(B * K), M, D)
    return _combine_fn(vals.reshape(B, K, D), w, next_pred)


# speedup vs baseline: 1.0295x; 1.0295x over previous
import functools

import jax
import jax.numpy as jnp
from jax import lax
from jax.experimental import pallas as pl
from jax.experimental.pallas import tpu as pltpu
from jax.experimental.pallas import tpu_sc as plsc

B = 1024
M = 100000
D = 256
K = 32
W = 128
MSTEP = 2048
NSTEP = (M + MSTEP - 1) // MSTEP
MPAD = NSTEP * MSTEP
NBLK = MPAD // W
GPB = MSTEP // W
NEG = -3.4e38
LAMBDAH = 0.3


def _score_body(q_ref, k_ref, s_ref, bm_ref):
    j = pl.program_id(0)
    q = q_ref[...]
    keys = k_ref[...]
    dots = lax.dot_general(q, keys, (((1,), (1,)), ((), ())),
                           preferred_element_type=jnp.float32)
    ones8 = jnp.ones((8, D), jnp.float32)
    s2 = lax.dot_general(ones8, keys * keys, (((1,), (1,)), ((), ())),
                         preferred_element_type=jnp.float32)[0:1]
    col = j * MSTEP + lax.broadcasted_iota(jnp.int32, (1, MSTEP), 1)
    score = dots - 0.5 * s2
    score = jnp.where(col >= M, NEG, score)
    s3 = score.reshape(B, GPB, W)
    s_ref[...] = s3
    bm_ref[...] = jnp.max(s3, axis=2)[None]


def _scores_fn(query, mem_keys, interpret=False):
    return pl.pallas_call(
        _score_body,
        grid=(NSTEP,),
        in_specs=[
            pl.BlockSpec((B, D), lambda j: (0, 0)),
            pl.BlockSpec((MSTEP, D), lambda j: (j, 0)),
        ],
        out_specs=[
            pl.BlockSpec((B, GPB, W), lambda j: (0, j, 0)),
            pl.BlockSpec((1, B, GPB), lambda j: (j, 0, 0)),
        ],
        out_shape=[
            jax.ShapeDtypeStruct((B, NBLK, W), jnp.float32),
            jax.ShapeDtypeStruct((NSTEP, B, GPB), jnp.float32),
        ],
        interpret=interpret,
    )(query, mem_keys)


def _topblocks_body(bm_ref, gidx_ref):
    bm = bm_ref[...]
    iota = lax.broadcasted_iota(jnp.int32, (B, NBLK), 1)
    row = lax.broadcasted_iota(jnp.int32, (B, 1), 0)
    for t in range(K):
        m = jnp.max(bm, axis=1, keepdims=True)
        pos = jnp.min(jnp.where(bm == m, iota, jnp.int32(1 << 30)),
                      axis=1, keepdims=True)
        bm = jnp.where(iota == pos, NEG, bm)
        gidx_ref[:, t:t + 1] = row * NBLK + pos
    del bm


def _topblocks_fn(bmax, interpret=False):
    return pl.pallas_call(
        _topblocks_body,
        out_shape=jax.ShapeDtypeStruct((B, K), jnp.int32),
        interpret=interpret,
    )(bmax)


def _sc_gather(table, idx, rows, d):
    n = idx.shape[0]
    info = plsc.get_sparse_core_info()
    nw = info.num_cores * info.num_subcores
    per_w = n // nw
    chunk = 128 if per_w % 128 == 0 else per_w
    nchunk = per_w // chunk
    mesh = plsc.VectorSubcoreMesh(core_axis_name="c", subcore_axis_name="s")

    @functools.partial(
        pl.kernel, mesh=mesh,
        out_type=jax.ShapeDtypeStruct((n, d), jnp.float32),
        scratch_types=[
            pltpu.VMEM((per_w,), jnp.int32),
            pltpu.VMEM((chunk, d), jnp.float32),
            pltpu.VMEM((chunk, d), jnp.float32),
            pltpu.SemaphoreType.DMA,
            pltpu.SemaphoreType.DMA,
        ],
    )
    def k(table_hbm, idx_hbm, out_hbm, idx_v, rows0, rows1, gsem, osem):
        wid = lax.axis_index("s") * info.num_cores + lax.axis_index("c")
        base = wid * per_w
        pltpu.sync_copy(idx_hbm.at[pl.ds(base, per_w)], idx_v)
        bufs = (rows0, rows1)

        def gstart(i, buf):
            return pltpu.async_copy(
                table_hbm.at[idx_v.at[pl.ds(i * chunk, chunk)]], buf, gsem)

        def ostart(i, buf):
            return pltpu.async_copy(
                buf, out_hbm.at[pl.ds(base + i * chunk, chunk)], osem)

        gops = [gstart(i, bufs[i % 2]) for i in range(min(2, nchunk))]
        oops = []
        for i in range(nchunk):
            gops[i].wait()
            oops.append(ostart(i, bufs[i % 2]))
            if i + 2 < nchunk:
                oops[i].wait()
                gops.append(gstart(i + 2, bufs[i % 2]))
        for o in oops[max(0, nchunk - 2):]:
            o.wait()

    return k(table, idx)


def _extract_body(c_ref, gidx_ref, q_ref, ti_ref, w_ref):
    c = c_ref[...]
    gidx = gidx_ref[...]
    q = q_ref[...]
    rb = c.shape[0]
    iota = lax.broadcasted_iota(jnp.int32, (rb, K * W), 1)
    kiota = lax.broadcasted_iota(jnp.int32, (rb, K), 1)
    row = lax.broadcasted_iota(jnp.int32, (rb, 1), 0)
    blkid = gidx - (pl.program_id(0) * rb + row) * NBLK
    scores = []
    for t in range(K):
        m = jnp.max(c, axis=1, keepdims=True)
        pos = jnp.min(jnp.where(c == m, iota, jnp.int32(1 << 30)),
                      axis=1, keepdims=True)
        c = jnp.where(iota == pos, NEG, c)
        g = jnp.right_shift(pos, W.bit_length() - 1)
        off = jnp.bitwise_and(pos, jnp.int32(W - 1))
        blk = jnp.sum(jnp.where(kiota == g, blkid, 0), axis=1, keepdims=True)
        ti_ref[:, t:t + 1] = blk * W + off
        scores.append(m)
    ts = jnp.concatenate(scores, axis=1)
    q2 = jnp.sum(q * q, axis=1, keepdims=True)
    sim = jnp.exp(ts - 0.5 * q2)
    w_ref[...] = sim / (jnp.sum(sim, axis=1, keepdims=True) + 1e-8)


def _extract_fn(cand, gidx, query, interpret=False):
    rb = 256
    nb = B // rb
    return pl.pallas_call(
        _extract_body,
        grid=(nb,),
        in_specs=[
            pl.BlockSpec((rb, K * W), lambda i: (i, 0)),
            pl.BlockSpec((rb, K), lambda i: (i, 0)),
            pl.BlockSpec((rb, D), lambda i: (i, 0)),
        ],
        out_specs=[
            pl.BlockSpec((rb, K), lambda i: (i, 0)),
            pl.BlockSpec((rb, K), lambda i: (i, 0)),
        ],
        out_shape=[
            jax.ShapeDtypeStruct((B, K), jnp.int32),
            jax.ShapeDtypeStruct((B, K), jnp.float32),
        ],
        interpret=interpret,
    )(cand, gidx, query)


def _combine_body(v_ref, w_ref, np_ref, o_ref):
    vals = v_ref[...]
    w = w_ref[...]
    pred = jnp.sum(vals * w[:, :, None], axis=1)
    o_ref[...] = LAMBDAH * pred + (1.0 - LAMBDAH) * np_ref[...]


def _combine_fn(vals, w, next_pred, interpret=False):
    rb = 128
    nb = B // rb
    return pl.pallas_call(
        _combine_body,
        grid=(nb,),
        in_specs=[
            pl.BlockSpec((rb, K, D), lambda i: (i, 0, 0)),
            pl.BlockSpec((rb, K), lambda i: (i, 0)),
            pl.BlockSpec((rb, D), lambda i: (i, 0)),
        ],
        out_specs=pl.BlockSpec((rb, D), lambda i: (i, 0)),
        out_shape=jax.ShapeDtypeStruct((B, D), jnp.float32),
        interpret=interpret,
    )(vals, w, next_pred)


def kernel(query, mem_keys, mem_values, next_pred, k):
    scores, bmax3 = _scores_fn(query, mem_keys)
    bmax = bmax3.transpose(1, 0, 2).reshape(B, NBLK)
    gidx = _topblocks_fn(bmax)
    cand = _sc_gather(scores.reshape(B * NBLK, W), gidx.reshape(B * K), B * NBLK, W)
    top_idx, w = _extract_fn(cand.reshape(B, K * W), gidx, query)
    vals = _sc_gather(mem_values, top_idx.reshape(B * K), M, D)
    return _combine_fn(vals.reshape(B, K, D), w, next_pred)
    bmax = bmax3.transpose(1, 0, 2).reshape(B, NBLK)
    gidx = _topblocks_fn(bmax)
    cand = _sc_gather(scores.reshape(B * NBLK, W), gidx.reshape(B * K), B * NBLK, W)
    top_idx, w = _extract_fn(cand.reshape(B, K * W), gidx, query)
    vals = _sc_gather(mem_values, top_idx.reshape(B * K), M, D)
    return _combine_fn(vals.reshape(B, K, D), w, next_pred)


# speedup vs baseline: 1.2423x; 1.2067x over previous
import functools

import jax
import jax.numpy as jnp
from jax import lax
from jax.experimental import pallas as pl
from jax.experimental.pallas import tpu as pltpu
from jax.experimental.pallas import tpu_sc as plsc

B = 1024
M = 100000
D = 256
K = 32
W = 128
MSTEP = 2048
NSTEP = (M + MSTEP - 1) // MSTEP
MPAD = NSTEP * MSTEP
NBLK = MPAD // W
GPB = MSTEP // W
NEG = -3.4e38
LAMBDAH = 0.3


def _score_body(q_ref, k_ref, s_ref, bm_ref):
    j = pl.program_id(0)
    q = q_ref[...]
    keys = k_ref[...]
    dots = lax.dot_general(q, keys, (((1,), (1,)), ((), ())),
                           preferred_element_type=jnp.float32)
    ones8 = jnp.ones((8, D), jnp.float32)
    s2 = lax.dot_general(ones8, keys * keys, (((1,), (1,)), ((), ())),
                         preferred_element_type=jnp.float32)[0:1]
    col = j * MSTEP + lax.broadcasted_iota(jnp.int32, (1, MSTEP), 1)
    score = dots - 0.5 * s2
    score = jnp.where(col >= M, NEG, score)
    s3 = score.reshape(B, GPB, W)
    s_ref[...] = s3
    bm_ref[...] = jnp.max(s3, axis=2)[None]


def _scores_fn(query, mem_keys, interpret=False):
    return pl.pallas_call(
        _score_body,
        grid=(NSTEP,),
        in_specs=[
            pl.BlockSpec((B, D), lambda j: (0, 0)),
            pl.BlockSpec((MSTEP, D), lambda j: (j, 0)),
        ],
        out_specs=[
            pl.BlockSpec((B, GPB, W), lambda j: (0, j, 0)),
            pl.BlockSpec((1, B, GPB), lambda j: (j, 0, 0)),
        ],
        out_shape=[
            jax.ShapeDtypeStruct((B, NBLK, W), jnp.float32),
            jax.ShapeDtypeStruct((NSTEP, B, GPB), jnp.float32),
        ],
        interpret=interpret,
    )(query, mem_keys)


def _topblocks_body(bm_ref, gidx_ref):
    bm = bm_ref[...]
    iota = lax.broadcasted_iota(jnp.int32, (B, NBLK), 1)
    row = lax.broadcasted_iota(jnp.int32, (B, 1), 0)
    for t in range(K):
        m = jnp.max(bm, axis=1, keepdims=True)
        pos = jnp.min(jnp.where(bm == m, iota, jnp.int32(1 << 30)),
                      axis=1, keepdims=True)
        bm = jnp.where(iota == pos, NEG, bm)
        gidx_ref[:, t:t + 1] = row * NBLK + pos
    del bm


def _topblocks_fn(bmax, interpret=False):
    return pl.pallas_call(
        _topblocks_body,
        out_shape=jax.ShapeDtypeStruct((B, K), jnp.int32),
        interpret=interpret,
    )(bmax)


def _sc_gather(table, idx, rows, d):
    n = idx.shape[0]
    info = plsc.get_sparse_core_info()
    nw = info.num_cores * info.num_subcores
    per_w = n // nw
    chunk = 128 if per_w % 128 == 0 else per_w
    nchunk = per_w // chunk
    mesh = plsc.VectorSubcoreMesh(core_axis_name="c", subcore_axis_name="s")

    @functools.partial(
        pl.kernel, mesh=mesh,
        out_type=jax.ShapeDtypeStruct((n, d), jnp.float32),
        scratch_types=[
            pltpu.VMEM((per_w,), jnp.int32),
            pltpu.VMEM((chunk, d), jnp.float32),
            pltpu.VMEM((chunk, d), jnp.float32),
            pltpu.SemaphoreType.DMA,
            pltpu.SemaphoreType.DMA,
        ],
    )
    def k(table_hbm, idx_hbm, out_hbm, idx_v, rows0, rows1, gsem, osem):
        wid = lax.axis_index("s") * info.num_cores + lax.axis_index("c")
        base = wid * per_w
        pltpu.sync_copy(idx_hbm.at[pl.ds(base, per_w)], idx_v)
        bufs = (rows0, rows1)

        def gstart(i, buf):
            return pltpu.async_copy(
                table_hbm.at[idx_v.at[pl.ds(i * chunk, chunk)]], buf, gsem)

        def ostart(i, buf):
            return pltpu.async_copy(
                buf, out_hbm.at[pl.ds(base + i * chunk, chunk)], osem)

        gops = [gstart(i, bufs[i % 2]) for i in range(min(2, nchunk))]
        oops = []
        for i in range(nchunk):
            gops[i].wait()
            oops.append(ostart(i, bufs[i % 2]))
            if i + 2 < nchunk:
                oops[i].wait()
                gops.append(gstart(i + 2, bufs[i % 2]))
        for o in oops[max(0, nchunk - 2):]:
            o.wait()

    return k(table, idx)


def _extract_body(c_ref, gidx_ref, q_ref, ti_ref, w_ref):
    c = c_ref[...]
    gidx = gidx_ref[...]
    q = q_ref[...]
    rb = c.shape[0]
    bits = lax.bitcast_convert_type(c, jnp.int32)
    mono = bits ^ jnp.bitwise_and(lax.shift_right_arithmetic(bits, 31),
                                  jnp.int32(0x7FFFFFFF))
    iota = lax.broadcasted_iota(jnp.int32, (rb, K * W), 1)
    key = jnp.bitwise_or(jnp.bitwise_and(mono, jnp.int32(-4096)), iota)
    kiota = lax.broadcasted_iota(jnp.int32, (rb, K), 1)
    row = lax.broadcasted_iota(jnp.int32, (rb, 1), 0)
    blkid = gidx - (pl.program_id(0) * rb + row) * NBLK
    scores = []
    for t in range(K):
        m = jnp.max(key, axis=1, keepdims=True)
        key = jnp.where(key == m, jnp.int32(-2147483648), key)
        pos = jnp.bitwise_and(m, jnp.int32(4095))
        g = jnp.right_shift(pos, W.bit_length() - 1)
        off = jnp.bitwise_and(pos, jnp.int32(W - 1))
        blk = jnp.sum(jnp.where(kiota == g, blkid, 0), axis=1, keepdims=True)
        ti_ref[:, t:t + 1] = blk * W + off
        scores.append(m - pos)
    tm = jnp.concatenate(scores, axis=1)
    sbits = tm ^ jnp.bitwise_and(lax.shift_right_arithmetic(tm, 31),
                                 jnp.int32(0x7FFFFFFF))
    ts = lax.bitcast_convert_type(sbits, jnp.float32)
    q2 = jnp.sum(q * q, axis=1, keepdims=True)
    sim = jnp.exp(ts - 0.5 * q2)
    w_ref[...] = sim / (jnp.sum(sim, axis=1, keepdims=True) + 1e-8)


def _extract_fn(cand, gidx, query, interpret=False):
    rb = 256
    nb = B // rb
    return pl.pallas_call(
        _extract_body,
        grid=(nb,),
        in_specs=[
            pl.BlockSpec((rb, K * W), lambda i: (i, 0)),
            pl.BlockSpec((rb, K), lambda i: (i, 0)),
            pl.BlockSpec((rb, D), lambda i: (i, 0)),
        ],
        out_specs=[
            pl.BlockSpec((rb, K), lambda i: (i, 0)),
            pl.BlockSpec((rb, K), lambda i: (i, 0)),
        ],
        out_shape=[
            jax.ShapeDtypeStruct((B, K), jnp.int32),
            jax.ShapeDtypeStruct((B, K), jnp.float32),
        ],
        interpret=interpret,
    )(cand, gidx, query)


def _combine_body(v_ref, w_ref, np_ref, o_ref):
    vals = v_ref[...]
    w = w_ref[...]
    pred = jnp.sum(vals * w[:, :, None], axis=1)
    o_ref[...] = LAMBDAH * pred + (1.0 - LAMBDAH) * np_ref[...]


def _combine_fn(vals, w, next_pred, interpret=False):
    rb = 128
    nb = B // rb
    return pl.pallas_call(
        _combine_body,
        grid=(nb,),
        in_specs=[
            pl.BlockSpec((rb, K, D), lambda i: (i, 0, 0)),
            pl.BlockSpec((rb, K), lambda i: (i, 0)),
            pl.BlockSpec((rb, D), lambda i: (i, 0)),
        ],
        out_specs=pl.BlockSpec((rb, D), lambda i: (i, 0)),
        out_shape=jax.ShapeDtypeStruct((B, D), jnp.float32),
        interpret=interpret,
    )(vals, w, next_pred)


def kernel(query, mem_keys, mem_values, next_pred, k):
    scores, bmax3 = _scores_fn(query, mem_keys)
    bmax = bmax3.transpose(1, 0, 2).reshape(B, NBLK)
    gidx = _topblocks_fn(bmax)
    cand = _sc_gather(scores.reshape(B * NBLK, W), gidx.reshape(B * K), B * NBLK, W)
    top_idx, w = _extract_fn(cand.reshape(B, K * W), gidx, query)
    vals = _sc_gather(mem_values, top_idx.reshape(B * K), M, D)
    return _combine_fn(vals.reshape(B, K, D), w, next_pred)
    bmax = bmax3.transpose(1, 0, 2).reshape(B, NBLK)
    gidx = _topblocks_fn(bmax)
    cand = _sc_gather(scores.reshape(B * NBLK, W), gidx.reshape(B * K), B * NBLK, W)
    top_idx, w = _extract_fn(cand.reshape(B, K * W), gidx, query)
    vals = _sc_gather(mem_values, top_idx.reshape(B * K), M, D)
    return _combine_fn(vals.reshape(B, K, D), w, next_pred)
